# Initial kernel scaffold; baseline (speedup 1.0000x reference)
import jax
import jax.numpy as jnp
from jax.experimental import pallas as pl


def kernel(x, weight):
    raise NotImplementedError("write your pallas kernel here")


# speedup vs baseline: 1.0481x; 1.0481x over previous
import functools

import jax
import jax.numpy as jnp
from jax.experimental import pallas as pl
from jax.experimental.pallas import tpu as pltpu

TOPK = 8
NUM_EXPERTS = 64
BLOCK_ROWS = 512


def _router_body(x_ref, wt_ref, w_out_ref, i_out_ref):
    scores = jnp.dot(x_ref[...], wt_ref[...],
                     preferred_element_type=jnp.float32)
    rows = scores.shape[0]
    col = jax.lax.broadcasted_iota(jnp.int32, (rows, NUM_EXPERTS), 1)

    vals = []
    idxs = []
    for _ in range(TOPK):
        m = jnp.max(scores, axis=1, keepdims=True)
        amax = jnp.min(jnp.where(scores == m, col, NUM_EXPERTS),
                       axis=1, keepdims=True)
        vals.append(m)
        idxs.append(amax)
        scores = jnp.where(col == amax, -jnp.inf, scores)

    top_vals = jnp.concatenate(vals, axis=1)
    top_idx = jnp.concatenate(idxs, axis=1)
    s = jax.nn.sigmoid(top_vals)
    w = s / jnp.sum(s, axis=1, keepdims=True)
    w_out_ref[...] = w
    i_out_ref[...] = top_idx


@jax.jit
def kernel(x, weight):
    n_tokens = x.shape[0]
    grid = (n_tokens // BLOCK_ROWS,)
    wt = weight.T
    w_out, i_out = pl.pallas_call(
        _router_body,
        grid=grid,
        in_specs=[
            pl.BlockSpec((BLOCK_ROWS, x.shape[1]), lambda i: (i, 0)),
            pl.BlockSpec((x.shape[1], NUM_EXPERTS), lambda i: (0, 0)),
        ],
        out_specs=[
            pl.BlockSpec((BLOCK_ROWS, TOPK), lambda i: (i, 0)),
            pl.BlockSpec((BLOCK_ROWS, TOPK), lambda i: (i, 0)),
        ],
        out_shape=[
            jax.ShapeDtypeStruct((n_tokens, TOPK), jnp.float32),
            jax.ShapeDtypeStruct((n_tokens, TOPK), jnp.int32),
        ],
    )(x, wt)
    return (w_out, i_out)
